# baseline (speedup 1.0000x reference)
import functools
import math

import jax
import jax.numpy as jnp
from jax import lax
from jax.experimental import pallas as pl
from jax.experimental.pallas import tpu as pltpu

N_NODES_C = 10000
N_EDGES_C = 320000
HID = 256

EDGE_BLK = 1280
NODE_BLK = 2000


def _edge_e_body(ea_ref, w1_ref, b1_ref, u_refs, c_refs, out_refs):
    ea = ea_ref[...]
    hid = jnp.maximum(
        jnp.dot(ea, w1_ref[...], preferred_element_type=jnp.float32)
        + b1_ref[...], 0.0)
    for u_ref, c_ref, o_ref in zip(u_refs, c_refs, out_refs):
        o_ref[...] = (jnp.dot(hid, u_ref[...], preferred_element_type=jnp.float32)
                      + c_ref[...])


def _edge_e(edge_attr, w1, b1, us, cs):
    n_edges = edge_attr.shape[0]
    grid = n_edges // EDGE_BLK

    def body(ea_ref, w1_ref, b1_ref, u0, u1, u2, u3, c0, c1, c2, c3,
             o0, o1, o2, o3):
        _edge_e_body(ea_ref, w1_ref, b1_ref, (u0, u1, u2, u3),
                     (c0, c1, c2, c3), (o0, o1, o2, o3))

    full = lambda shape: pl.BlockSpec(shape, lambda i: (0,) * len(shape))
    in_specs = [pl.BlockSpec((EDGE_BLK, 16), lambda i: (i, 0)),
                full(w1.shape), full(b1.shape)]
    in_specs += [full(u.shape) for u in us]
    in_specs += [full(c.shape) for c in cs]
    out_specs = [pl.BlockSpec((EDGE_BLK, u.shape[1]), lambda i: (i, 0))
                 for u in us]
    out_shape = [jax.ShapeDtypeStruct((n_edges, u.shape[1]), jnp.float32)
                 for u in us]
    return pl.pallas_call(
        body, grid=(grid,), in_specs=in_specs, out_specs=out_specs,
        out_shape=out_shape)(edge_attr, w1, b1, *us, *cs)


def _node_mlp(h, agg, w1, b1, w2, b2, scale, shift):
    n = h.shape[0]
    grid = n // NODE_BLK

    def body(h_ref, a_ref, w1_ref, b1_ref, w2_ref, b2_ref, s_ref, t_ref, o_ref):
        z = h_ref[...] + a_ref[...]
        y = jnp.maximum(
            jnp.dot(z, w1_ref[...], preferred_element_type=jnp.float32)
            + b1_ref[...], 0.0)
        y = jnp.dot(y, w2_ref[...], preferred_element_type=jnp.float32) + b2_ref[...]
        y = jnp.maximum(y, 0.0)
        o_ref[...] = y * s_ref[...] + t_ref[...]

    full = lambda shape: pl.BlockSpec(shape, lambda i: (0,) * len(shape))
    in_dim = h.shape[1]
    in_specs = [pl.BlockSpec((NODE_BLK, in_dim), lambda i: (i, 0)),
                pl.BlockSpec((NODE_BLK, in_dim), lambda i: (i, 0)),
                full(w1.shape), full(b1.shape), full(w2.shape), full(b2.shape),
                full(scale.shape), full(shift.shape)]
    return pl.pallas_call(
        body, grid=(grid,), in_specs=in_specs,
        out_specs=pl.BlockSpec((NODE_BLK, HID), lambda i: (i, 0)),
        out_shape=jax.ShapeDtypeStruct((n, HID), jnp.float32),
    )(h, agg, w1, b1, w2, b2, scale, shift)


def _tail(h, batch2d, lattice, lw1, lb1, ls, lt, lw2, lb2,
          fw1, fb1, fs, ft, fw2, fb2, ngraphs):
    n = h.shape[0]

    def body(h_ref, b_ref, lat_ref, lw1_ref, lb1_ref, ls_ref, lt_ref,
             lw2_ref, lb2_ref, fw1_ref, fb1_ref, fs_ref, ft_ref,
             fw2_ref, fb2_ref, o_ref):
        b = b_ref[...]
        gids = lax.broadcasted_iota(jnp.int32, (ngraphs, n), 0)
        onehot = (gids == jnp.broadcast_to(b, (ngraphs, n))).astype(jnp.float32)
        sums = jnp.dot(onehot, h_ref[...], preferred_element_type=jnp.float32)
        cnt = jnp.sum(onehot, axis=1, keepdims=True)
        pool = sums / jnp.maximum(cnt, 1.0)
        lat = lat_ref[...]
        lf = jnp.maximum(
            jnp.dot(lat, lw1_ref[...], preferred_element_type=jnp.float32)
            + lb1_ref[...], 0.0)
        lf = lf * ls_ref[...] + lt_ref[...]
        lf = jnp.dot(lf, lw2_ref[...], preferred_element_type=jnp.float32) + lb2_ref[...]
        cat = jnp.concatenate([pool, lf], axis=1)
        y = jnp.maximum(
            jnp.dot(cat, fw1_ref[...], preferred_element_type=jnp.float32)
            + fb1_ref[...], 0.0)
        y = y * fs_ref[...] + ft_ref[...]
        o_ref[...] = (jnp.dot(y, fw2_ref[...], preferred_element_type=jnp.float32)
                      + fb2_ref[...])

    args = (h, batch2d, lattice, lw1, lb1, ls, lt, lw2, lb2,
            fw1, fb1, fs, ft, fw2, fb2)
    return pl.pallas_call(
        body,
        out_shape=jax.ShapeDtypeStruct((ngraphs, fw2.shape[1]), jnp.float32),
    )(*args)


def kernel(x, edge_attr, lattice, params, edge_index, batch):
    p = params
    num_gnn = 4
    bn_scale = 1.0 / math.sqrt(1.0 + 1e-5)

    us, cs = [], []
    for i in range(num_gnn):
        we = p[f"g{i}_We"]
        us.append(p["ee_W2"] @ we)
        cs.append(p["ee_b2"] @ we + p[f"g{i}_be"])

    es = _edge_e(edge_attr, p["ee_W1"], p["ee_b1"], us, cs)

    src, dst = edge_index[0], edge_index[1]
    h = x
    for i in range(num_gnn):
        m = jnp.maximum(h[src] + es[i], 0.0)
        agg = jax.ops.segment_sum(m, dst, num_segments=N_NODES_C)
        h = _node_mlp(h, agg, p[f"g{i}_W1"], p[f"g{i}_b1"],
                      p[f"g{i}_W2"], p[f"g{i}_b2"],
                      p[f"g{i}_g"] * bn_scale, p[f"g{i}_bt"])

    ngraphs = lattice.shape[0]
    out = _tail(h, batch.reshape(1, -1), lattice.reshape(ngraphs, 9),
                p["lat_W1"], p["lat_b1"], p["lat_g"] * bn_scale, p["lat_bt"],
                p["lat_W2"], p["lat_b2"],
                p["f_W1"], p["f_b1"], p["f_g"] * bn_scale, p["f_bt"],
                p["f_W2"], p["f_b2"], ngraphs)
    return out


# speedup vs baseline: 4.3021x; 4.3021x over previous
import functools
import math

import jax
import jax.numpy as jnp
from jax import lax
from jax.experimental import pallas as pl
from jax.experimental.pallas import tpu as pltpu
from jax.experimental.pallas import tpu_sc as plsc

N_NODES_C = 10000
N_PAD = 10240
N_EDGES_C = 320000
HID = 256

EDGE_BLK = 1280
NODE_BLK = 2048

N_TILES = 16
C_FSPLIT = 80
C_ESPLIT = 40
G_STAGE = 10
NS = 25
ROWS_PER_TILE = N_PAD // N_TILES


def _edge_e(edge_attr, w1, b1, us, cs):
    n_edges = edge_attr.shape[0]
    grid = n_edges // EDGE_BLK

    def body(ea_ref, w1_ref, b1_ref, u0, u1, u2, u3, c0, c1, c2, c3,
             o0, o1, o2, o3):
        ea = ea_ref[...]
        hid = jnp.maximum(
            jnp.dot(ea, w1_ref[...], preferred_element_type=jnp.float32)
            + b1_ref[...], 0.0)
        for u_ref, c_ref, o_ref in zip((u0, u1, u2, u3), (c0, c1, c2, c3),
                                       (o0, o1, o2, o3)):
            e = (jnp.dot(hid, u_ref[...], preferred_element_type=jnp.float32)
                 + c_ref[...])
            if o_ref.shape[0] == 2:
                f = e.shape[1] // 2
                o_ref[0] = e[:, :f]
                o_ref[1] = e[:, f:]
            else:
                o_ref[...] = e

    full = lambda shape: pl.BlockSpec(shape, lambda i: (0,) * len(shape))
    in_specs = [pl.BlockSpec((EDGE_BLK, 16), lambda i: (i, 0)),
                full(w1.shape), full(b1.shape)]
    in_specs += [full(u.shape) for u in us]
    in_specs += [full(c.shape) for c in cs]
    out_specs = [pl.BlockSpec((EDGE_BLK, us[0].shape[1]), lambda i: (i, 0))]
    out_specs += [pl.BlockSpec((2, EDGE_BLK, u.shape[1] // 2),
                               lambda i: (0, i, 0)) for u in us[1:]]
    out_shape = [jax.ShapeDtypeStruct((n_edges, us[0].shape[1]), jnp.float32)]
    out_shape += [jax.ShapeDtypeStruct((2, n_edges, u.shape[1] // 2), jnp.float32)
                  for u in us[1:]]
    return pl.pallas_call(
        body, grid=(grid,), in_specs=in_specs, out_specs=out_specs,
        out_shape=out_shape)(edge_attr, w1, b1, *us, *cs)


def _sc_agg(h_in, e_in, src3, dst3, esplit):
    f = h_in.shape[-1]
    nf16 = f // 16
    c_sz = C_ESPLIT if esplit else C_FSPLIT
    n_chunk = NS * G_STAGE
    mesh = plsc.VectorSubcoreMesh(core_axis_name="c", subcore_axis_name="s")

    @functools.partial(
        pl.kernel,
        out_type=jax.ShapeDtypeStruct((2, N_PAD, f), jnp.float32),
        mesh=mesh,
        scratch_types=[
            pltpu.VMEM((G_STAGE, 1, c_sz), jnp.int32),
            pltpu.VMEM((G_STAGE, 1, c_sz), jnp.int32),
            pltpu.VMEM((G_STAGE, 1, c_sz), jnp.int32),
            pltpu.VMEM((G_STAGE, 1, c_sz), jnp.int32),
            pltpu.VMEM((c_sz, f), jnp.float32),
            pltpu.VMEM((c_sz, f), jnp.float32),
            pltpu.VMEM((c_sz, f), jnp.float32),
            pltpu.VMEM((c_sz, f), jnp.float32),
            pltpu.VMEM_SHARED((N_PAD, f), jnp.float32),
            pltpu.SemaphoreType.DMA,
            pltpu.SemaphoreType.DMA,
            pltpu.SemaphoreType.DMA,
            pltpu.SemaphoreType.DMA,
            pltpu.SemaphoreType.DMA,
            pltpu.SemaphoreType.DMA,
        ],
    )
    def k(h_hbm, e_hbm, src_hbm, dst_hbm, out_hbm,
          ss0, ss1, ds0, ds1, h0, h1, e0, e1, aggs,
          sm0, sm1, hs0, hs1, es0, es1):
        cid = lax.axis_index("c")
        sid = lax.axis_index("s")
        sstg = (ss0, ss1)
        dstg = (ds0, ds1)
        hbufs = (h0, h1)
        ebufs = (e0, e1)
        ssems = (sm0, sm1)
        hsems = (hs0, hs1)
        esems = (es0, es1)
        if esplit:
            h_view = h_hbm
            e_view = e_hbm
            wid = cid * N_TILES + sid
        else:
            h_view = h_hbm.at[cid]
            e_view = e_hbm.at[cid]
            wid = sid
        cbase = wid * n_chunk

        zeros16 = jnp.zeros((16,), jnp.float32)

        def zrow(j, carry):
            for ff in range(nf16):
                e0[j, pl.ds(ff * 16, 16)] = zeros16
            return carry

        lax.fori_loop(0, c_sz, zrow, 0)
        row0 = sid * ROWS_PER_TILE

        def zcopy(q, carry):
            pltpu.sync_copy(e0, aggs.at[pl.ds(row0 + q * c_sz, c_sz)])
            return carry

        lax.fori_loop(0, ROWS_PER_TILE // c_sz, zcopy, 0)
        plsc.subcore_barrier()

        def issue_stage(si, sb):
            @pl.when(si < NS)
            def _():
                off = cbase + si * G_STAGE
                pltpu.async_copy(src_hbm.at[pl.ds(off, G_STAGE)],
                                 sstg[sb], ssems[sb])
                pltpu.async_copy(dst_hbm.at[pl.ds(off, G_STAGE)],
                                 dstg[sb], ssems[sb])

        def wait_stage(si, sb):
            off = cbase + si * G_STAGE
            pltpu.make_async_copy(src_hbm.at[pl.ds(off, G_STAGE)],
                                  sstg[sb], ssems[sb]).wait()
            pltpu.make_async_copy(dst_hbm.at[pl.ds(off, G_STAGE)],
                                  dstg[sb], ssems[sb]).wait()

        def issue_data(kk, g, sb, db):
            pltpu.async_copy(h_view.at[sstg[sb].at[g, 0]], hbufs[db],
                             hsems[db])
            pltpu.async_copy(e_view.at[pl.ds((cbase + kk) * c_sz, c_sz)],
                             ebufs[db], esems[db])

        def consume_data(kk, g, sb, db):
            pltpu.make_async_copy(h_view.at[sstg[sb].at[g, 0]], hbufs[db],
                                  hsems[db]).wait()
            pltpu.make_async_copy(
                e_view.at[pl.ds((cbase + kk) * c_sz, c_sz)], ebufs[db],
                esems[db]).wait()
            hb, eb = hbufs[db], ebufs[db]

            def ew(j, carry):
                for ff in range(nf16):
                    sl = pl.ds(ff * 16, 16)
                    eb[j, sl] = jnp.maximum(hb[j, sl] + eb[j, sl], 0.0)
                return carry

            lax.fori_loop(0, c_sz, ew, 0)
            pltpu.sync_copy(eb, aggs.at[dstg[sb].at[g, 0]], add=True)

        def emit_stage(si, sb):
            wait_stage(si, sb)
            issue_stage(si + 1, 1 - sb)
            k0 = si * G_STAGE
            issue_data(k0, 0, sb, 0)

            def gp(t, carry):
                g0 = 2 * t
                issue_data(k0 + g0 + 1, g0 + 1, sb, 1)
                consume_data(k0 + g0, g0, sb, 0)

                @pl.when(g0 + 2 < G_STAGE)
                def _():
                    issue_data(k0 + g0 + 2, g0 + 2, sb, 0)

                consume_data(k0 + g0 + 1, g0 + 1, sb, 1)
                return carry

            lax.fori_loop(0, G_STAGE // 2, gp, 0)

        issue_stage(0, 0)

        def pair(t, carry):
            emit_stage(2 * t, 0)
            emit_stage(2 * t + 1, 1)
            return carry

        lax.fori_loop(0, NS // 2, pair, 0)
        if NS % 2:
            emit_stage(NS - 1, 0)

        plsc.subcore_barrier()
        pltpu.sync_copy(aggs.at[pl.ds(row0, ROWS_PER_TILE)],
                        out_hbm.at[cid, pl.ds(row0, ROWS_PER_TILE)])

    return k(h_in, e_in, src3, dst3)


def _node_mlp(h_arr, agg_split, w1, b1, w2, b2, scale, shift, esplit):
    n = agg_split.shape[1]
    fin = agg_split.shape[2]
    grid = n // NODE_BLK

    def body(h_ref, a_ref, w1_ref, b1_ref, w2_ref, b2_ref, s_ref, t_ref, o_ref):
        if esplit:
            z = h_ref[...] + a_ref[0] + a_ref[1]
        else:
            z = jnp.concatenate([h_ref[0] + a_ref[0], h_ref[1] + a_ref[1]],
                                axis=1)
        y = jnp.maximum(
            jnp.dot(z, w1_ref[...], preferred_element_type=jnp.float32)
            + b1_ref[...], 0.0)
        y = jnp.dot(y, w2_ref[...], preferred_element_type=jnp.float32) + b2_ref[...]
        y = jnp.maximum(y, 0.0)
        y = y * s_ref[...] + t_ref[...]
        o_ref[0] = y[:, :HID // 2]
        o_ref[1] = y[:, HID // 2:]

    full = lambda shape: pl.BlockSpec(shape, lambda i: (0,) * len(shape))
    h_spec = (pl.BlockSpec((NODE_BLK, fin), lambda i: (i, 0)) if esplit
              else pl.BlockSpec((2, NODE_BLK, fin), lambda i: (0, i, 0)))
    in_specs = [h_spec,
                pl.BlockSpec((2, NODE_BLK, fin), lambda i: (0, i, 0)),
                full(w1.shape), full(b1.shape), full(w2.shape), full(b2.shape),
                full(scale.shape), full(shift.shape)]
    return pl.pallas_call(
        body, grid=(grid,), in_specs=in_specs,
        out_specs=pl.BlockSpec((2, NODE_BLK, HID // 2), lambda i: (0, i, 0)),
        out_shape=jax.ShapeDtypeStruct((2, n, HID // 2), jnp.float32),
    )(h_arr, agg_split, w1, b1, w2, b2, scale, shift)


def _tail(h_split, batch2d, lattice, lw1, lb1, ls, lt, lw2, lb2,
          fw1, fb1, fs, ft, fw2, fb2, ngraphs):
    n = h_split.shape[1]

    def body(h_ref, b_ref, lat_ref, lw1_ref, lb1_ref, ls_ref, lt_ref,
             lw2_ref, lb2_ref, fw1_ref, fb1_ref, fs_ref, ft_ref,
             fw2_ref, fb2_ref, o_ref):
        h = jnp.concatenate([h_ref[0], h_ref[1]], axis=1)
        b = b_ref[...]
        gids = lax.broadcasted_iota(jnp.int32, (ngraphs, n), 0)
        onehot = (gids == jnp.broadcast_to(b, (ngraphs, n))).astype(jnp.float32)
        sums = jnp.dot(onehot, h, preferred_element_type=jnp.float32)
        cnt = jnp.sum(onehot, axis=1, keepdims=True)
        pool = sums / jnp.maximum(cnt, 1.0)
        lat = lat_ref[...]
        lf = jnp.maximum(
            jnp.dot(lat, lw1_ref[...], preferred_element_type=jnp.float32)
            + lb1_ref[...], 0.0)
        lf = lf * ls_ref[...] + lt_ref[...]
        lf = jnp.dot(lf, lw2_ref[...], preferred_element_type=jnp.float32) + lb2_ref[...]
        cat = jnp.concatenate([pool, lf], axis=1)
        y = jnp.maximum(
            jnp.dot(cat, fw1_ref[...], preferred_element_type=jnp.float32)
            + fb1_ref[...], 0.0)
        y = y * fs_ref[...] + ft_ref[...]
        o_ref[...] = (jnp.dot(y, fw2_ref[...], preferred_element_type=jnp.float32)
                      + fb2_ref[...])

    args = (h_split, batch2d, lattice, lw1, lb1, ls, lt, lw2, lb2,
            fw1, fb1, fs, ft, fw2, fb2)
    return pl.pallas_call(
        body,
        out_shape=jax.ShapeDtypeStruct((ngraphs, fw2.shape[1]), jnp.float32),
    )(*args)


def kernel(x, edge_attr, lattice, params, edge_index, batch):
    p = params
    num_gnn = 4
    bn_scale = 1.0 / math.sqrt(1.0 + 1e-5)

    us, cs = [], []
    for i in range(num_gnn):
        we = p[f"g{i}_We"]
        us.append(p["ee_W2"] @ we)
        cs.append(p["ee_b2"] @ we + p[f"g{i}_be"])

    es = _edge_e(edge_attr, p["ee_W1"], p["ee_b1"], us, cs)

    src16 = edge_index[0].reshape(-1, 1, C_FSPLIT)
    dst16 = edge_index[1].reshape(-1, 1, C_FSPLIT)
    src32 = edge_index[0].reshape(-1, 1, C_ESPLIT)
    dst32 = edge_index[1].reshape(-1, 1, C_ESPLIT)

    xp = jnp.pad(x, ((0, N_PAD - x.shape[0]), (0, 0)))
    agg0 = _sc_agg(xp, es[0], src32, dst32, esplit=True)
    h_split = _node_mlp(xp, agg0, p["g0_W1"], p["g0_b1"],
                        p["g0_W2"], p["g0_b2"],
                        p["g0_g"] * bn_scale, p["g0_bt"], esplit=True)
    for i in range(1, num_gnn):
        agg_split = _sc_agg(h_split, es[i], src16, dst16, esplit=False)
        h_split = _node_mlp(h_split, agg_split, p[f"g{i}_W1"], p[f"g{i}_b1"],
                            p[f"g{i}_W2"], p[f"g{i}_b2"],
                            p[f"g{i}_g"] * bn_scale, p[f"g{i}_bt"],
                            esplit=False)

    ngraphs = lattice.shape[0]
    batch_pad = jnp.pad(batch, (0, N_PAD - batch.shape[0]),
                        constant_values=ngraphs)
    out = _tail(h_split, batch_pad.reshape(1, -1), lattice.reshape(ngraphs, 9),
                p["lat_W1"], p["lat_b1"], p["lat_g"] * bn_scale, p["lat_bt"],
                p["lat_W2"], p["lat_b2"],
                p["f_W1"], p["f_b1"], p["f_g"] * bn_scale, p["f_bt"],
                p["f_W2"], p["f_b2"], ngraphs)
    return out


# speedup vs baseline: 4.4380x; 1.0316x over previous
import functools
import math

import jax
import jax.numpy as jnp
from jax import lax
from jax.experimental import pallas as pl
from jax.experimental.pallas import tpu as pltpu
from jax.experimental.pallas import tpu_sc as plsc

N_NODES_C = 10000
N_PAD = 10240
N_EDGES_C = 320000
HID = 256

EDGE_BLK = 1280
NODE_BLK = 2048

N_TILES = 16
C_FSPLIT = 80
C_ESPLIT = 40
G_STAGE = 10
NS = 25
ROWS_PER_TILE = N_PAD // N_TILES


def _edge_e(edge_attr, w1, b1, u, c, split):
    n_edges = edge_attr.shape[0]
    grid = n_edges // EDGE_BLK
    fdim = u.shape[1]

    def body(ea_ref, w1_ref, b1_ref, u_ref, c_ref, o_ref):
        ea = ea_ref[...]
        hid = jnp.maximum(
            jnp.dot(ea, w1_ref[...], preferred_element_type=jnp.float32)
            + b1_ref[...], 0.0)
        e = (jnp.dot(hid, u_ref[...], preferred_element_type=jnp.float32)
             + c_ref[...])
        if split:
            o_ref[0] = e[:, :fdim // 2]
            o_ref[1] = e[:, fdim // 2:]
        else:
            o_ref[...] = e

    full = lambda shape: pl.BlockSpec(shape, lambda i: (0,) * len(shape))
    in_specs = [pl.BlockSpec((EDGE_BLK, 16), lambda i: (i, 0)),
                full(w1.shape), full(b1.shape), full(u.shape), full(c.shape)]
    if split:
        out_specs = pl.BlockSpec((2, EDGE_BLK, fdim // 2), lambda i: (0, i, 0))
        out_shape = jax.ShapeDtypeStruct((2, n_edges, fdim // 2), jnp.float32)
    else:
        out_specs = pl.BlockSpec((EDGE_BLK, fdim), lambda i: (i, 0))
        out_shape = jax.ShapeDtypeStruct((n_edges, fdim), jnp.float32)
    return pl.pallas_call(
        body, grid=(grid,), in_specs=in_specs, out_specs=out_specs,
        out_shape=out_shape)(edge_attr, w1, b1, u, c)


def _sc_agg(h_in, e_in, src3, dst3, esplit):
    f = h_in.shape[-1]
    nf16 = f // 16
    c_sz = C_ESPLIT if esplit else C_FSPLIT
    n_chunk = NS * G_STAGE
    mesh = plsc.VectorSubcoreMesh(core_axis_name="c", subcore_axis_name="s")

    @functools.partial(
        pl.kernel,
        out_type=jax.ShapeDtypeStruct((2, N_PAD, f), jnp.float32),
        mesh=mesh,
        scratch_types=[
            pltpu.VMEM((G_STAGE, 1, c_sz), jnp.int32),
            pltpu.VMEM((G_STAGE, 1, c_sz), jnp.int32),
            pltpu.VMEM((G_STAGE, 1, c_sz), jnp.int32),
            pltpu.VMEM((G_STAGE, 1, c_sz), jnp.int32),
            pltpu.VMEM((c_sz, f), jnp.float32),
            pltpu.VMEM((c_sz, f), jnp.float32),
            pltpu.VMEM((c_sz, f), jnp.float32),
            pltpu.VMEM((c_sz, f), jnp.float32),
            pltpu.VMEM_SHARED((N_PAD, f), jnp.float32),
            pltpu.SemaphoreType.DMA,
            pltpu.SemaphoreType.DMA,
            pltpu.SemaphoreType.DMA,
            pltpu.SemaphoreType.DMA,
            pltpu.SemaphoreType.DMA,
            pltpu.SemaphoreType.DMA,
        ],
    )
    def k(h_hbm, e_hbm, src_hbm, dst_hbm, out_hbm,
          ss0, ss1, ds0, ds1, h0, h1, e0, e1, aggs,
          sm0, sm1, hs0, hs1, es0, es1):
        cid = lax.axis_index("c")
        sid = lax.axis_index("s")
        sstg = (ss0, ss1)
        dstg = (ds0, ds1)
        hbufs = (h0, h1)
        ebufs = (e0, e1)
        ssems = (sm0, sm1)
        hsems = (hs0, hs1)
        esems = (es0, es1)
        if esplit:
            h_view = h_hbm
            e_view = e_hbm
            wid = cid * N_TILES + sid
        else:
            h_view = h_hbm.at[cid]
            e_view = e_hbm.at[cid]
            wid = sid
        cbase = wid * n_chunk

        zeros16 = jnp.zeros((16,), jnp.float32)

        def zrow(j, carry):
            for ff in range(nf16):
                e0[j, pl.ds(ff * 16, 16)] = zeros16
            return carry

        lax.fori_loop(0, c_sz, zrow, 0)
        row0 = sid * ROWS_PER_TILE

        def zcopy(q, carry):
            pltpu.sync_copy(e0, aggs.at[pl.ds(row0 + q * c_sz, c_sz)])
            return carry

        lax.fori_loop(0, ROWS_PER_TILE // c_sz, zcopy, 0)
        plsc.subcore_barrier()

        def issue_stage(si, sb):
            @pl.when(si < NS)
            def _():
                off = cbase + si * G_STAGE
                pltpu.async_copy(src_hbm.at[pl.ds(off, G_STAGE)],
                                 sstg[sb], ssems[sb])
                pltpu.async_copy(dst_hbm.at[pl.ds(off, G_STAGE)],
                                 dstg[sb], ssems[sb])

        def wait_stage(si, sb):
            off = cbase + si * G_STAGE
            pltpu.make_async_copy(src_hbm.at[pl.ds(off, G_STAGE)],
                                  sstg[sb], ssems[sb]).wait()
            pltpu.make_async_copy(dst_hbm.at[pl.ds(off, G_STAGE)],
                                  dstg[sb], ssems[sb]).wait()

        def issue_data(kk, g, sb, db):
            pltpu.async_copy(h_view.at[sstg[sb].at[g, 0]], hbufs[db],
                             hsems[db])
            pltpu.async_copy(e_view.at[pl.ds((cbase + kk) * c_sz, c_sz)],
                             ebufs[db], esems[db])

        def consume_data(kk, g, sb, db):
            pltpu.make_async_copy(h_view.at[sstg[sb].at[g, 0]], hbufs[db],
                                  hsems[db]).wait()
            pltpu.make_async_copy(
                e_view.at[pl.ds((cbase + kk) * c_sz, c_sz)], ebufs[db],
                esems[db]).wait()
            hb, eb = hbufs[db], ebufs[db]

            def ew(j, carry):
                for ff in range(nf16):
                    sl = pl.ds(ff * 16, 16)
                    eb[j, sl] = jnp.maximum(hb[j, sl] + eb[j, sl], 0.0)
                return carry

            lax.fori_loop(0, c_sz, ew, 0)
            pltpu.sync_copy(eb, aggs.at[dstg[sb].at[g, 0]], add=True)

        def emit_stage(si, sb):
            wait_stage(si, sb)
            issue_stage(si + 1, 1 - sb)
            k0 = si * G_STAGE
            issue_data(k0, 0, sb, 0)

            def gp(t, carry):
                g0 = 2 * t
                issue_data(k0 + g0 + 1, g0 + 1, sb, 1)
                consume_data(k0 + g0, g0, sb, 0)

                @pl.when(g0 + 2 < G_STAGE)
                def _():
                    issue_data(k0 + g0 + 2, g0 + 2, sb, 0)

                consume_data(k0 + g0 + 1, g0 + 1, sb, 1)
                return carry

            lax.fori_loop(0, G_STAGE // 2, gp, 0)

        issue_stage(0, 0)

        def pair(t, carry):
            emit_stage(2 * t, 0)
            emit_stage(2 * t + 1, 1)
            return carry

        lax.fori_loop(0, NS // 2, pair, 0)
        if NS % 2:
            emit_stage(NS - 1, 0)

        plsc.subcore_barrier()
        pltpu.sync_copy(aggs.at[pl.ds(row0, ROWS_PER_TILE)],
                        out_hbm.at[cid, pl.ds(row0, ROWS_PER_TILE)])

    return k(h_in, e_in, src3, dst3)


def _node_mlp(h_arr, agg_split, w1, b1, w2, b2, scale, shift, esplit):
    n = agg_split.shape[1]
    fin = agg_split.shape[2]
    grid = n // NODE_BLK

    def body(h_ref, a_ref, w1_ref, b1_ref, w2_ref, b2_ref, s_ref, t_ref,
             o_ref):
        if esplit:
            z = h_ref[...] + a_ref[0] + a_ref[1]
        else:
            z = jnp.concatenate([h_ref[0] + a_ref[0], h_ref[1] + a_ref[1]],
                                axis=1)
        y = jnp.maximum(
            jnp.dot(z, w1_ref[...], preferred_element_type=jnp.float32)
            + b1_ref[...], 0.0)
        y = jnp.dot(y, w2_ref[...], preferred_element_type=jnp.float32) + b2_ref[...]
        y = jnp.maximum(y, 0.0)
        y = y * s_ref[...] + t_ref[...]
        o_ref[0] = y[:, :HID // 2]
        o_ref[1] = y[:, HID // 2:]

    full = lambda shape: pl.BlockSpec(shape, lambda i: (0,) * len(shape))
    h_spec = (pl.BlockSpec((NODE_BLK, fin), lambda i: (i, 0)) if esplit
              else pl.BlockSpec((2, NODE_BLK, fin), lambda i: (0, i, 0)))
    in_specs = [h_spec,
                pl.BlockSpec((2, NODE_BLK, fin), lambda i: (0, i, 0)),
                full(w1.shape), full(b1.shape), full(w2.shape), full(b2.shape),
                full(scale.shape), full(shift.shape)]
    return pl.pallas_call(
        body, grid=(grid,), in_specs=in_specs,
        out_specs=pl.BlockSpec((2, NODE_BLK, HID // 2), lambda i: (0, i, 0)),
        out_shape=jax.ShapeDtypeStruct((2, n, HID // 2), jnp.float32),
    )(h_arr, agg_split, w1, b1, w2, b2, scale, shift)


def _tail(h_split, batch2d, lattice, lw1, lb1, ls, lt, lw2, lb2,
          fw1, fb1, fs, ft, fw2, fb2, ngraphs):
    n = h_split.shape[1]

    def body(h_ref, b_ref, lat_ref, lw1_ref, lb1_ref, ls_ref, lt_ref,
             lw2_ref, lb2_ref, fw1_ref, fb1_ref, fs_ref, ft_ref,
             fw2_ref, fb2_ref, o_ref):
        h = jnp.concatenate([h_ref[0], h_ref[1]], axis=1)
        b = b_ref[...]
        gids = lax.broadcasted_iota(jnp.int32, (ngraphs, n), 0)
        onehot = (gids == jnp.broadcast_to(b, (ngraphs, n))).astype(jnp.float32)
        sums = jnp.dot(onehot, h, preferred_element_type=jnp.float32)
        cnt = jnp.sum(onehot, axis=1, keepdims=True)
        pool = sums / jnp.maximum(cnt, 1.0)
        lat = lat_ref[...]
        lf = jnp.maximum(
            jnp.dot(lat, lw1_ref[...], preferred_element_type=jnp.float32)
            + lb1_ref[...], 0.0)
        lf = lf * ls_ref[...] + lt_ref[...]
        lf = jnp.dot(lf, lw2_ref[...], preferred_element_type=jnp.float32) + lb2_ref[...]
        cat = jnp.concatenate([pool, lf], axis=1)
        y = jnp.maximum(
            jnp.dot(cat, fw1_ref[...], preferred_element_type=jnp.float32)
            + fb1_ref[...], 0.0)
        y = y * fs_ref[...] + ft_ref[...]
        o_ref[...] = (jnp.dot(y, fw2_ref[...], preferred_element_type=jnp.float32)
                      + fb2_ref[...])

    args = (h_split, batch2d, lattice, lw1, lb1, ls, lt, lw2, lb2,
            fw1, fb1, fs, ft, fw2, fb2)
    return pl.pallas_call(
        body,
        out_shape=jax.ShapeDtypeStruct((ngraphs, fw2.shape[1]), jnp.float32),
    )(*args)


def kernel(x, edge_attr, lattice, params, edge_index, batch):
    p = params
    num_gnn = 4
    bn_scale = 1.0 / math.sqrt(1.0 + 1e-5)

    us, cs = [], []
    for i in range(num_gnn):
        we = p[f"g{i}_We"]
        us.append(p["ee_W2"] @ we)
        cs.append(p["ee_b2"] @ we + p[f"g{i}_be"])

    src16 = edge_index[0].reshape(-1, 1, C_FSPLIT)
    dst16 = edge_index[1].reshape(-1, 1, C_FSPLIT)
    src32 = edge_index[0].reshape(-1, 1, C_ESPLIT)
    dst32 = edge_index[1].reshape(-1, 1, C_ESPLIT)

    xp = jnp.pad(x, ((0, N_PAD - x.shape[0]), (0, 0)))
    e0 = _edge_e(edge_attr, p["ee_W1"], p["ee_b1"], us[0], cs[0], split=False)
    agg0 = _sc_agg(xp, e0, src32, dst32, esplit=True)
    h_split = _node_mlp(xp, agg0, p["g0_W1"], p["g0_b1"],
                        p["g0_W2"], p["g0_b2"],
                        p["g0_g"] * bn_scale, p["g0_bt"], esplit=True)
    for i in range(1, num_gnn):
        e_i = _edge_e(edge_attr, p["ee_W1"], p["ee_b1"], us[i], cs[i],
                      split=True)
        agg_split = _sc_agg(h_split, e_i, src16, dst16, esplit=False)
        h_split = _node_mlp(h_split, agg_split,
                            p[f"g{i}_W1"], p[f"g{i}_b1"],
                            p[f"g{i}_W2"], p[f"g{i}_b2"],
                            p[f"g{i}_g"] * bn_scale, p[f"g{i}_bt"],
                            esplit=False)

    ngraphs = lattice.shape[0]
    batch_pad = jnp.pad(batch, (0, N_PAD - batch.shape[0]),
                        constant_values=ngraphs)
    out = _tail(h_split, batch_pad.reshape(1, -1), lattice.reshape(ngraphs, 9),
                p["lat_W1"], p["lat_b1"], p["lat_g"] * bn_scale, p["lat_bt"],
                p["lat_W2"], p["lat_b2"],
                p["f_W1"], p["f_b1"], p["f_g"] * bn_scale, p["f_bt"],
                p["f_W2"], p["f_b2"], ngraphs)
    return out
